# baseline (speedup 1.0000x reference)
import functools

import jax
import jax.numpy as jnp
from jax import lax
from jax.experimental import pallas as pl
from jax.experimental.pallas import tpu as pltpu
from jax.experimental.pallas import tpu_sc as plsc

NUM_PATCHES = 8192
PROJ_DIM = 1024

SC_ROWS = 2048
TC_ROWS = NUM_PATCHES - SC_ROWS

SC_CHUNK = 256
SC_NBUF = 2

TC_BLOCK = 2048
MERGE_BLOCK = 2048


@functools.lru_cache(maxsize=None)
def _make_sc_gather():
    info = plsc.get_sparse_core_info()
    nc = info.num_cores
    rows_per_c = SC_ROWS // nc
    n_ch = rows_per_c // SC_CHUNK

    mesh = plsc.ScalarSubcoreMesh(axis_name="c", num_cores=nc)

    @functools.partial(
        pl.kernel,
        out_type=jax.ShapeDtypeStruct((SC_ROWS, PROJ_DIM), jnp.float32),
        mesh=mesh,
        scratch_types=(
            [pltpu.VMEM_SHARED((SC_NBUF * SC_CHUNK, PROJ_DIM), jnp.float32)]
            + [pltpu.SemaphoreType.DMA] * (2 * SC_NBUF)
        ),
    )
    def gather_rows(table_hbm, piece_hbm, shared, *sems):
        sin = sems[:SC_NBUF]
        sout = sems[SC_NBUF:]
        cid = lax.axis_index("c")
        src_base = TC_ROWS + cid * rows_per_c
        dst_base = cid * rows_per_c

        def in_copy(i):
            b = i % SC_NBUF
            return pltpu.async_copy(
                table_hbm.at[pl.ds(src_base + i * SC_CHUNK, SC_CHUNK)],
                shared.at[pl.ds(b * SC_CHUNK, SC_CHUNK)], sin[b])

        def out_copy(i):
            b = i % SC_NBUF
            return pltpu.async_copy(
                shared.at[pl.ds(b * SC_CHUNK, SC_CHUNK)],
                piece_hbm.at[pl.ds(dst_base + i * SC_CHUNK, SC_CHUNK)],
                sout[b])

        h_in = [None] * n_ch
        h_out = [None] * n_ch
        h_in[0] = in_copy(0)
        for i in range(n_ch):
            if i + 1 < n_ch:
                if i + 1 - SC_NBUF >= 0:
                    h_out[i + 1 - SC_NBUF].wait()
                h_in[i + 1] = in_copy(i + 1)
            h_in[i].wait()
            h_out[i] = out_copy(i)
        for j in range(max(0, n_ch - SC_NBUF), n_ch):
            h_out[j].wait()

    return gather_rows


def _copy_body(src_ref, out_ref):
    out_ref[...] = src_ref[...]


@functools.lru_cache(maxsize=None)
def _make_tc_bulk():
    return pl.pallas_call(
        _copy_body,
        grid=(TC_ROWS // TC_BLOCK,),
        in_specs=[pl.BlockSpec((TC_BLOCK, PROJ_DIM), lambda i: (i, 0))],
        out_specs=pl.BlockSpec((TC_BLOCK, PROJ_DIM), lambda i: (i, 0)),
        out_shape=jax.ShapeDtypeStruct((NUM_PATCHES, PROJ_DIM), jnp.float32),
    )


def _merge_body(piece_ref, full_ref, out_ref):
    del full_ref
    out_ref[...] = piece_ref[...]


@functools.lru_cache(maxsize=None)
def _make_tc_merge():
    base = TC_ROWS // MERGE_BLOCK
    return pl.pallas_call(
        _merge_body,
        grid=(SC_ROWS // MERGE_BLOCK,),
        in_specs=[
            pl.BlockSpec((MERGE_BLOCK, PROJ_DIM), lambda i: (i, 0)),
            pl.BlockSpec(memory_space=pltpu.MemorySpace.HBM),
        ],
        out_specs=pl.BlockSpec((MERGE_BLOCK, PROJ_DIM), lambda i: (base + i, 0)),
        out_shape=jax.ShapeDtypeStruct((NUM_PATCHES, PROJ_DIM), jnp.float32),
        input_output_aliases={1: 0},
    )


def kernel(tokens, table):
    del tokens
    sc_piece = _make_sc_gather()(table)
    tc_full = _make_tc_bulk()(table)
    out = _make_tc_merge()(sc_piece, tc_full)
    return out[None]


# speedup vs baseline: 1.0150x; 1.0150x over previous
import functools

import jax
import jax.numpy as jnp
from jax import lax
from jax.experimental import pallas as pl
from jax.experimental.pallas import tpu as pltpu
from jax.experimental.pallas import tpu_sc as plsc

NUM_PATCHES = 8192
PROJ_DIM = 1024

SC_ROWS = 1024
TC_ROWS = NUM_PATCHES - SC_ROWS

SC_CHUNK = 256
SC_NBUF = 2

TC_BLOCK = 1024
MERGE_BLOCK = 1024


@functools.lru_cache(maxsize=None)
def _make_sc_gather():
    info = plsc.get_sparse_core_info()
    nc = info.num_cores
    rows_per_c = SC_ROWS // nc
    n_ch = rows_per_c // SC_CHUNK

    mesh = plsc.ScalarSubcoreMesh(axis_name="c", num_cores=nc)

    @functools.partial(
        pl.kernel,
        out_type=jax.ShapeDtypeStruct((SC_ROWS, PROJ_DIM), jnp.float32),
        mesh=mesh,
        scratch_types=(
            [pltpu.VMEM_SHARED((SC_NBUF * SC_CHUNK, PROJ_DIM), jnp.float32)]
            + [pltpu.SemaphoreType.DMA] * (2 * SC_NBUF)
        ),
    )
    def gather_rows(table_hbm, piece_hbm, shared, *sems):
        sin = sems[:SC_NBUF]
        sout = sems[SC_NBUF:]
        cid = lax.axis_index("c")
        src_base = TC_ROWS + cid * rows_per_c
        dst_base = cid * rows_per_c

        def in_copy(i):
            b = i % SC_NBUF
            return pltpu.async_copy(
                table_hbm.at[pl.ds(src_base + i * SC_CHUNK, SC_CHUNK)],
                shared.at[pl.ds(b * SC_CHUNK, SC_CHUNK)], sin[b])

        def out_copy(i):
            b = i % SC_NBUF
            return pltpu.async_copy(
                shared.at[pl.ds(b * SC_CHUNK, SC_CHUNK)],
                piece_hbm.at[pl.ds(dst_base + i * SC_CHUNK, SC_CHUNK)],
                sout[b])

        h_in = [None] * n_ch
        h_out = [None] * n_ch
        h_in[0] = in_copy(0)
        for i in range(n_ch):
            if i + 1 < n_ch:
                if i + 1 - SC_NBUF >= 0:
                    h_out[i + 1 - SC_NBUF].wait()
                h_in[i + 1] = in_copy(i + 1)
            h_in[i].wait()
            h_out[i] = out_copy(i)
        for j in range(max(0, n_ch - SC_NBUF), n_ch):
            h_out[j].wait()

    return gather_rows


def _copy_body(src_ref, out_ref):
    out_ref[...] = src_ref[...]


@functools.lru_cache(maxsize=None)
def _make_tc_bulk():
    return pl.pallas_call(
        _copy_body,
        grid=(TC_ROWS // TC_BLOCK,),
        in_specs=[pl.BlockSpec((TC_BLOCK, PROJ_DIM), lambda i: (i, 0))],
        out_specs=pl.BlockSpec((TC_BLOCK, PROJ_DIM), lambda i: (i, 0)),
        out_shape=jax.ShapeDtypeStruct((NUM_PATCHES, PROJ_DIM), jnp.float32),
    )


def _merge_body(piece_ref, full_ref, out_ref):
    del full_ref
    out_ref[...] = piece_ref[...]


@functools.lru_cache(maxsize=None)
def _make_tc_merge():
    base = TC_ROWS // MERGE_BLOCK
    return pl.pallas_call(
        _merge_body,
        grid=(SC_ROWS // MERGE_BLOCK,),
        in_specs=[
            pl.BlockSpec((MERGE_BLOCK, PROJ_DIM), lambda i: (i, 0)),
            pl.BlockSpec(memory_space=pltpu.MemorySpace.HBM),
        ],
        out_specs=pl.BlockSpec((MERGE_BLOCK, PROJ_DIM), lambda i: (base + i, 0)),
        out_shape=jax.ShapeDtypeStruct((NUM_PATCHES, PROJ_DIM), jnp.float32),
        input_output_aliases={1: 0},
    )


def kernel(tokens, table):
    del tokens
    sc_piece = _make_sc_gather()(table)
    tc_full = _make_tc_bulk()(table)
    out = _make_tc_merge()(sc_piece, tc_full)
    return out[None]


# speedup vs baseline: 1.0337x; 1.0184x over previous
import functools

import jax
import jax.numpy as jnp
from jax import lax
from jax.experimental import pallas as pl
from jax.experimental.pallas import tpu as pltpu
from jax.experimental.pallas import tpu_sc as plsc

NUM_PATCHES = 8192
PROJ_DIM = 1024

SC_ROWS = 1024
TC_ROWS = NUM_PATCHES - SC_ROWS

SC_CHUNK = 256
SC_NBUF = 2

TC_BLOCK = 1792
MERGE_BLOCK = 1024


@functools.lru_cache(maxsize=None)
def _make_sc_gather():
    info = plsc.get_sparse_core_info()
    nc = info.num_cores
    rows_per_c = SC_ROWS // nc
    n_ch = rows_per_c // SC_CHUNK

    mesh = plsc.ScalarSubcoreMesh(axis_name="c", num_cores=nc)

    @functools.partial(
        pl.kernel,
        out_type=jax.ShapeDtypeStruct((SC_ROWS, PROJ_DIM), jnp.float32),
        mesh=mesh,
        scratch_types=(
            [pltpu.VMEM_SHARED((SC_NBUF * SC_CHUNK, PROJ_DIM), jnp.float32)]
            + [pltpu.SemaphoreType.DMA] * (2 * SC_NBUF)
        ),
    )
    def gather_rows(table_hbm, piece_hbm, shared, *sems):
        sin = sems[:SC_NBUF]
        sout = sems[SC_NBUF:]
        cid = lax.axis_index("c")
        src_base = TC_ROWS + cid * rows_per_c
        dst_base = cid * rows_per_c

        def in_copy(i):
            b = i % SC_NBUF
            return pltpu.async_copy(
                table_hbm.at[pl.ds(src_base + i * SC_CHUNK, SC_CHUNK)],
                shared.at[pl.ds(b * SC_CHUNK, SC_CHUNK)], sin[b])

        def out_copy(i):
            b = i % SC_NBUF
            return pltpu.async_copy(
                shared.at[pl.ds(b * SC_CHUNK, SC_CHUNK)],
                piece_hbm.at[pl.ds(dst_base + i * SC_CHUNK, SC_CHUNK)],
                sout[b])

        h_in = [None] * n_ch
        h_out = [None] * n_ch
        h_in[0] = in_copy(0)
        for i in range(n_ch):
            if i + 1 < n_ch:
                if i + 1 - SC_NBUF >= 0:
                    h_out[i + 1 - SC_NBUF].wait()
                h_in[i + 1] = in_copy(i + 1)
            h_in[i].wait()
            h_out[i] = out_copy(i)
        for j in range(max(0, n_ch - SC_NBUF), n_ch):
            h_out[j].wait()

    return gather_rows


def _copy_body(src_ref, out_ref):
    out_ref[...] = src_ref[...]


@functools.lru_cache(maxsize=None)
def _make_tc_bulk():
    return pl.pallas_call(
        _copy_body,
        grid=(TC_ROWS // TC_BLOCK,),
        in_specs=[pl.BlockSpec((TC_BLOCK, PROJ_DIM), lambda i: (i, 0))],
        out_specs=pl.BlockSpec((TC_BLOCK, PROJ_DIM), lambda i: (i, 0)),
        out_shape=jax.ShapeDtypeStruct((NUM_PATCHES, PROJ_DIM), jnp.float32),
    )


def _merge_body(piece_ref, full_ref, out_ref):
    del full_ref
    out_ref[...] = piece_ref[...]


@functools.lru_cache(maxsize=None)
def _make_tc_merge():
    base = TC_ROWS // MERGE_BLOCK
    return pl.pallas_call(
        _merge_body,
        grid=(SC_ROWS // MERGE_BLOCK,),
        in_specs=[
            pl.BlockSpec((MERGE_BLOCK, PROJ_DIM), lambda i: (i, 0)),
            pl.BlockSpec(memory_space=pltpu.MemorySpace.HBM),
        ],
        out_specs=pl.BlockSpec((MERGE_BLOCK, PROJ_DIM), lambda i: (base + i, 0)),
        out_shape=jax.ShapeDtypeStruct((NUM_PATCHES, PROJ_DIM), jnp.float32),
        input_output_aliases={1: 0},
    )


def kernel(tokens, table):
    del tokens
    sc_piece = _make_sc_gather()(table)
    tc_full = _make_tc_bulk()(table)
    out = _make_tc_merge()(sc_piece, tc_full)
    return out[None]


# speedup vs baseline: 1.0725x; 1.0376x over previous
import functools

import jax
import jax.numpy as jnp
from jax import lax
from jax.experimental import pallas as pl
from jax.experimental.pallas import tpu as pltpu
from jax.experimental.pallas import tpu_sc as plsc

NUM_PATCHES = 8192
PROJ_DIM = 1024

SC_ROWS = 512
TC_ROWS = NUM_PATCHES - SC_ROWS

SC_CHUNK = 256
SC_NBUF = 2

TC_BLOCK = 1920
MERGE_BLOCK = 512


@functools.lru_cache(maxsize=None)
def _make_sc_gather():
    info = plsc.get_sparse_core_info()
    nc = info.num_cores
    rows_per_c = SC_ROWS // nc
    n_ch = rows_per_c // SC_CHUNK

    mesh = plsc.ScalarSubcoreMesh(axis_name="c", num_cores=nc)

    @functools.partial(
        pl.kernel,
        out_type=jax.ShapeDtypeStruct((SC_ROWS, PROJ_DIM), jnp.float32),
        mesh=mesh,
        scratch_types=(
            [pltpu.VMEM_SHARED((SC_NBUF * SC_CHUNK, PROJ_DIM), jnp.float32)]
            + [pltpu.SemaphoreType.DMA] * (2 * SC_NBUF)
        ),
    )
    def gather_rows(table_hbm, piece_hbm, shared, *sems):
        sin = sems[:SC_NBUF]
        sout = sems[SC_NBUF:]
        cid = lax.axis_index("c")
        src_base = TC_ROWS + cid * rows_per_c
        dst_base = cid * rows_per_c

        def in_copy(i):
            b = i % SC_NBUF
            return pltpu.async_copy(
                table_hbm.at[pl.ds(src_base + i * SC_CHUNK, SC_CHUNK)],
                shared.at[pl.ds(b * SC_CHUNK, SC_CHUNK)], sin[b])

        def out_copy(i):
            b = i % SC_NBUF
            return pltpu.async_copy(
                shared.at[pl.ds(b * SC_CHUNK, SC_CHUNK)],
                piece_hbm.at[pl.ds(dst_base + i * SC_CHUNK, SC_CHUNK)],
                sout[b])

        h_in = [None] * n_ch
        h_out = [None] * n_ch
        h_in[0] = in_copy(0)
        for i in range(n_ch):
            if i + 1 < n_ch:
                if i + 1 - SC_NBUF >= 0:
                    h_out[i + 1 - SC_NBUF].wait()
                h_in[i + 1] = in_copy(i + 1)
            h_in[i].wait()
            h_out[i] = out_copy(i)
        for j in range(max(0, n_ch - SC_NBUF), n_ch):
            h_out[j].wait()

    return gather_rows


def _copy_body(src_ref, out_ref):
    out_ref[...] = src_ref[...]


@functools.lru_cache(maxsize=None)
def _make_tc_bulk():
    return pl.pallas_call(
        _copy_body,
        grid=(TC_ROWS // TC_BLOCK,),
        in_specs=[pl.BlockSpec((TC_BLOCK, PROJ_DIM), lambda i: (i, 0))],
        out_specs=pl.BlockSpec((TC_BLOCK, PROJ_DIM), lambda i: (i, 0)),
        out_shape=jax.ShapeDtypeStruct((NUM_PATCHES, PROJ_DIM), jnp.float32),
    )


def _merge_body(piece_ref, full_ref, out_ref):
    del full_ref
    out_ref[...] = piece_ref[...]


@functools.lru_cache(maxsize=None)
def _make_tc_merge():
    base = TC_ROWS // MERGE_BLOCK
    return pl.pallas_call(
        _merge_body,
        grid=(SC_ROWS // MERGE_BLOCK,),
        in_specs=[
            pl.BlockSpec((MERGE_BLOCK, PROJ_DIM), lambda i: (i, 0)),
            pl.BlockSpec(memory_space=pltpu.MemorySpace.HBM),
        ],
        out_specs=pl.BlockSpec((MERGE_BLOCK, PROJ_DIM), lambda i: (base + i, 0)),
        out_shape=jax.ShapeDtypeStruct((NUM_PATCHES, PROJ_DIM), jnp.float32),
        input_output_aliases={1: 0},
    )


def kernel(tokens, table):
    del tokens
    sc_piece = _make_sc_gather()(table)
    tc_full = _make_tc_bulk()(table)
    out = _make_tc_merge()(sc_piece, tc_full)
    return out[None]


# speedup vs baseline: 1.0982x; 1.0239x over previous
import functools

import jax
import jax.numpy as jnp
from jax import lax
from jax.experimental import pallas as pl
from jax.experimental.pallas import tpu as pltpu
from jax.experimental.pallas import tpu_sc as plsc

NUM_PATCHES = 8192
PROJ_DIM = 1024

SC_ROWS = 256
TC_ROWS = NUM_PATCHES - SC_ROWS

SC_CHUNK = 128
SC_NBUF = 2

TC_BLOCK = 1984
MERGE_BLOCK = 256


@functools.lru_cache(maxsize=None)
def _make_sc_gather():
    info = plsc.get_sparse_core_info()
    nc = info.num_cores
    rows_per_c = SC_ROWS // nc
    n_ch = rows_per_c // SC_CHUNK

    mesh = plsc.ScalarSubcoreMesh(axis_name="c", num_cores=nc)

    @functools.partial(
        pl.kernel,
        out_type=jax.ShapeDtypeStruct((SC_ROWS, PROJ_DIM), jnp.float32),
        mesh=mesh,
        scratch_types=(
            [pltpu.VMEM_SHARED((SC_NBUF * SC_CHUNK, PROJ_DIM), jnp.float32)]
            + [pltpu.SemaphoreType.DMA] * (2 * SC_NBUF)
        ),
    )
    def gather_rows(table_hbm, piece_hbm, shared, *sems):
        sin = sems[:SC_NBUF]
        sout = sems[SC_NBUF:]
        cid = lax.axis_index("c")
        src_base = TC_ROWS + cid * rows_per_c
        dst_base = cid * rows_per_c

        def in_copy(i):
            b = i % SC_NBUF
            return pltpu.async_copy(
                table_hbm.at[pl.ds(src_base + i * SC_CHUNK, SC_CHUNK)],
                shared.at[pl.ds(b * SC_CHUNK, SC_CHUNK)], sin[b])

        def out_copy(i):
            b = i % SC_NBUF
            return pltpu.async_copy(
                shared.at[pl.ds(b * SC_CHUNK, SC_CHUNK)],
                piece_hbm.at[pl.ds(dst_base + i * SC_CHUNK, SC_CHUNK)],
                sout[b])

        h_in = [None] * n_ch
        h_out = [None] * n_ch
        h_in[0] = in_copy(0)
        for i in range(n_ch):
            if i + 1 < n_ch:
                if i + 1 - SC_NBUF >= 0:
                    h_out[i + 1 - SC_NBUF].wait()
                h_in[i + 1] = in_copy(i + 1)
            h_in[i].wait()
            h_out[i] = out_copy(i)
        for j in range(max(0, n_ch - SC_NBUF), n_ch):
            h_out[j].wait()

    return gather_rows


def _copy_body(src_ref, out_ref):
    out_ref[...] = src_ref[...]


@functools.lru_cache(maxsize=None)
def _make_tc_bulk():
    return pl.pallas_call(
        _copy_body,
        grid=(TC_ROWS // TC_BLOCK,),
        in_specs=[pl.BlockSpec((TC_BLOCK, PROJ_DIM), lambda i: (i, 0))],
        out_specs=pl.BlockSpec((TC_BLOCK, PROJ_DIM), lambda i: (i, 0)),
        out_shape=jax.ShapeDtypeStruct((NUM_PATCHES, PROJ_DIM), jnp.float32),
    )


def _merge_body(piece_ref, full_ref, out_ref):
    del full_ref
    out_ref[...] = piece_ref[...]


@functools.lru_cache(maxsize=None)
def _make_tc_merge():
    base = TC_ROWS // MERGE_BLOCK
    return pl.pallas_call(
        _merge_body,
        grid=(SC_ROWS // MERGE_BLOCK,),
        in_specs=[
            pl.BlockSpec((MERGE_BLOCK, PROJ_DIM), lambda i: (i, 0)),
            pl.BlockSpec(memory_space=pltpu.MemorySpace.HBM),
        ],
        out_specs=pl.BlockSpec((MERGE_BLOCK, PROJ_DIM), lambda i: (base + i, 0)),
        out_shape=jax.ShapeDtypeStruct((NUM_PATCHES, PROJ_DIM), jnp.float32),
        input_output_aliases={1: 0},
    )


def kernel(tokens, table):
    del tokens
    sc_piece = _make_sc_gather()(table)
    tc_full = _make_tc_bulk()(table)
    out = _make_tc_merge()(sc_piece, tc_full)
    return out[None]


# speedup vs baseline: 1.1248x; 1.0243x over previous
import functools

import jax
import jax.numpy as jnp
from jax import lax
from jax.experimental import pallas as pl
from jax.experimental.pallas import tpu as pltpu
from jax.experimental.pallas import tpu_sc as plsc

NUM_PATCHES = 8192
PROJ_DIM = 1024

SC_ROWS = 128
TC_ROWS = NUM_PATCHES - SC_ROWS

SC_CHUNK = 64
SC_NBUF = 2

TC_BLOCK = 2016
MERGE_BLOCK = 128


@functools.lru_cache(maxsize=None)
def _make_sc_gather():
    info = plsc.get_sparse_core_info()
    nc = info.num_cores
    rows_per_c = SC_ROWS // nc
    n_ch = rows_per_c // SC_CHUNK

    mesh = plsc.ScalarSubcoreMesh(axis_name="c", num_cores=nc)

    @functools.partial(
        pl.kernel,
        out_type=jax.ShapeDtypeStruct((SC_ROWS, PROJ_DIM), jnp.float32),
        mesh=mesh,
        scratch_types=(
            [pltpu.VMEM_SHARED((SC_NBUF * SC_CHUNK, PROJ_DIM), jnp.float32)]
            + [pltpu.SemaphoreType.DMA] * (2 * SC_NBUF)
        ),
    )
    def gather_rows(table_hbm, piece_hbm, shared, *sems):
        sin = sems[:SC_NBUF]
        sout = sems[SC_NBUF:]
        cid = lax.axis_index("c")
        src_base = TC_ROWS + cid * rows_per_c
        dst_base = cid * rows_per_c

        def in_copy(i):
            b = i % SC_NBUF
            return pltpu.async_copy(
                table_hbm.at[pl.ds(src_base + i * SC_CHUNK, SC_CHUNK)],
                shared.at[pl.ds(b * SC_CHUNK, SC_CHUNK)], sin[b])

        def out_copy(i):
            b = i % SC_NBUF
            return pltpu.async_copy(
                shared.at[pl.ds(b * SC_CHUNK, SC_CHUNK)],
                piece_hbm.at[pl.ds(dst_base + i * SC_CHUNK, SC_CHUNK)],
                sout[b])

        h_in = [None] * n_ch
        h_out = [None] * n_ch
        h_in[0] = in_copy(0)
        for i in range(n_ch):
            if i + 1 < n_ch:
                if i + 1 - SC_NBUF >= 0:
                    h_out[i + 1 - SC_NBUF].wait()
                h_in[i + 1] = in_copy(i + 1)
            h_in[i].wait()
            h_out[i] = out_copy(i)
        for j in range(max(0, n_ch - SC_NBUF), n_ch):
            h_out[j].wait()

    return gather_rows


def _copy_body(src_ref, out_ref):
    out_ref[...] = src_ref[...]


@functools.lru_cache(maxsize=None)
def _make_tc_bulk():
    return pl.pallas_call(
        _copy_body,
        grid=(TC_ROWS // TC_BLOCK,),
        in_specs=[pl.BlockSpec((TC_BLOCK, PROJ_DIM), lambda i: (i, 0))],
        out_specs=pl.BlockSpec((TC_BLOCK, PROJ_DIM), lambda i: (i, 0)),
        out_shape=jax.ShapeDtypeStruct((NUM_PATCHES, PROJ_DIM), jnp.float32),
    )


def _merge_body(piece_ref, full_ref, out_ref):
    del full_ref
    out_ref[...] = piece_ref[...]


@functools.lru_cache(maxsize=None)
def _make_tc_merge():
    base = TC_ROWS // MERGE_BLOCK
    return pl.pallas_call(
        _merge_body,
        grid=(SC_ROWS // MERGE_BLOCK,),
        in_specs=[
            pl.BlockSpec((MERGE_BLOCK, PROJ_DIM), lambda i: (i, 0)),
            pl.BlockSpec(memory_space=pltpu.MemorySpace.HBM),
        ],
        out_specs=pl.BlockSpec((MERGE_BLOCK, PROJ_DIM), lambda i: (base + i, 0)),
        out_shape=jax.ShapeDtypeStruct((NUM_PATCHES, PROJ_DIM), jnp.float32),
        input_output_aliases={1: 0},
    )


def kernel(tokens, table):
    del tokens
    sc_piece = _make_sc_gather()(table)
    tc_full = _make_tc_bulk()(table)
    out = _make_tc_merge()(sc_piece, tc_full)
    return out[None]
